# baseline (speedup 1.0000x reference)
import functools

import jax
import jax.numpy as jnp
from jax import lax
from jax.experimental import pallas as pl
from jax.experimental.pallas import tpu as pltpu
from jax.experimental.pallas import tpu_sc as plsc

B = 16384
F = 4096
L = 16

B_TC = 8192
B_SC = B - B_TC

_info = plsc.get_sparse_core_info()
NC = _info.num_cores
NS = _info.num_subcores
NW = NC * NS

ROWS_PER_W = B_SC // NW
LD_PER_W = B // NW
RBLK = 4
NBUF = 2
NBLK = ROWS_PER_W // RBLK
NCHUNK = F // L

_mesh = plsc.VectorSubcoreMesh(core_axis_name="c", subcore_axis_name="s")


@functools.partial(
    pl.kernel,
    mesh=_mesh,
    compiler_params=pltpu.CompilerParams(needs_layout_passes=False),
    out_type=(
        jax.ShapeDtypeStruct((B * F,), jnp.float32),
        jax.ShapeDtypeStruct((B,), jnp.float32),
    ),
    scratch_types=[
        pltpu.VMEM((F,), jnp.int32),
        pltpu.VMEM((RBLK * F,), jnp.float32),
        pltpu.VMEM((RBLK * F,), jnp.float32),
        pltpu.VMEM((RBLK * F,), jnp.float32),
        pltpu.VMEM((RBLK * F,), jnp.float32),
        pltpu.VMEM((LD_PER_W,), jnp.float32),
        pltpu.SemaphoreType.DMA,
        pltpu.SemaphoreType.DMA,
        pltpu.SemaphoreType.DMA,
        pltpu.SemaphoreType.DMA,
    ],
)
def _sc_permute(x_hbm, perm_hbm, out_hbm, ld_hbm, perm_v,
                inb0, inb1, outb0, outb1, ldb, is0, is1, os0, os1):
    wid = lax.axis_index("s") * NC + lax.axis_index("c")
    base = B_TC + wid * ROWS_PER_W
    inbufs, outbufs = [inb0, inb1], [outb0, outb1]
    isems, osems = [is0, is1], [os0, os1]

    pltpu.sync_copy(perm_hbm, perm_v)

    zero = jnp.zeros((L,), jnp.float32)

    def zero_body(t, carry):
        ldb[pl.ds(t * L, L)] = zero
        return carry

    lax.fori_loop(0, LD_PER_W // L, zero_body, 0)
    pltpu.sync_copy(ldb, ld_hbm.at[pl.ds(wid * LD_PER_W, LD_PER_W)])

    def in_slice(gb):
        return x_hbm.at[pl.ds((base + gb * RBLK) * F, RBLK * F)]

    def out_slice(gb):
        return out_hbm.at[pl.ds((base + gb * RBLK) * F, RBLK * F)]

    for b in range(NBUF):
        pltpu.async_copy(in_slice(b), inbufs[b], isems[b])

    def block_body(g, carry):
        for b in range(NBUF):
            gb = g * NBUF + b
            pltpu.make_async_copy(in_slice(gb), inbufs[b], isems[b]).wait()

            @pl.when(gb >= NBUF)
            def _wait_out():
                pltpu.make_async_copy(
                    outbufs[b], out_slice(gb), osems[b]).wait()

            inb, outb = inbufs[b], outbufs[b]

            @plsc.parallel_loop(0, NCHUNK, unroll=4)
            def chunk_body(jc):
                j0 = jc * L
                pc = perm_v[pl.ds(j0, L)]
                for r in range(RBLK):
                    idx = pc + jnp.full((L,), r * F, jnp.int32)
                    outb[pl.ds(r * F + j0, L)] = plsc.load_gather(inb, [idx])

            pltpu.async_copy(outb, out_slice(gb), osems[b])

            @pl.when(gb + NBUF < NBLK)
            def _next_in():
                pltpu.async_copy(in_slice(gb + NBUF), inbufs[b], isems[b])
        return carry

    lax.fori_loop(0, NBLK // NBUF, block_body, 0)

    for b in range(NBUF):
        pltpu.make_async_copy(
            outbufs[b], out_slice(NBLK - NBUF + b), osems[b]).wait()


def _build_p_body(perm_ref, p_ref):
    rows = lax.broadcasted_iota(jnp.int32, (F, F), 0)
    cols = perm_ref[...][None, :]
    p_ref[...] = jnp.where(rows == cols, 1.0, 0.0).astype(jnp.bfloat16)


_build_p = pl.pallas_call(
    _build_p_body,
    out_shape=jax.ShapeDtypeStruct((F, F), jnp.bfloat16),
)

MBLK = 256


def _tc_matmul_body(x_ref, p_ref, out_ref):
    out_ref[...] = jnp.dot(
        x_ref[...].astype(jnp.bfloat16), p_ref[...],
        preferred_element_type=jnp.float32)


_tc_matmul = pl.pallas_call(
    _tc_matmul_body,
    grid=(B_TC // MBLK,),
    in_specs=[
        pl.BlockSpec((MBLK, F), lambda i: (i, 0)),
        pl.BlockSpec((F, F), lambda i: (0, 0)),
    ],
    out_specs=pl.BlockSpec((MBLK, F), lambda i: (i, 0)),
    out_shape=jax.ShapeDtypeStruct((B_TC, F), jnp.float32),
)


def kernel(x, perm, inv_perm):
    del inv_perm
    perm_i32 = perm.astype(jnp.int32)
    out_flat, log_det = _sc_permute(x.reshape(-1), perm_i32)
    p = _build_p(perm_i32)
    tc_rows = _tc_matmul(x[:B_TC], p)
    out = lax.dynamic_update_slice(out_flat.reshape(B, F), tc_rows, (0, 0))
    return (out, log_det)


# speedup vs baseline: 1.0505x; 1.0505x over previous
import functools

import jax
import jax.numpy as jnp
from jax import lax
from jax.experimental import pallas as pl
from jax.experimental.pallas import tpu as pltpu
from jax.experimental.pallas import tpu_sc as plsc

B = 16384
F = 4096
L = 16

B_TC = 8192
B_SC = B - B_TC

_info = plsc.get_sparse_core_info()
NC = _info.num_cores
NS = _info.num_subcores
NW = NC * NS

ROWS_PER_W = B_SC // NW
LD_PER_W = B // NW
RBLK = 4
NBUF = 2
NBLK = ROWS_PER_W // RBLK
NCHUNK = F // L

_mesh = plsc.VectorSubcoreMesh(core_axis_name="c", subcore_axis_name="s")


@functools.partial(
    pl.kernel,
    mesh=_mesh,
    compiler_params=pltpu.CompilerParams(needs_layout_passes=False),
    out_type=(
        jax.ShapeDtypeStruct((B_SC * F,), jnp.float32),
        jax.ShapeDtypeStruct((B,), jnp.float32),
    ),
    scratch_types=[
        pltpu.VMEM((F,), jnp.int32),
        pltpu.VMEM((RBLK * F,), jnp.float32),
        pltpu.VMEM((RBLK * F,), jnp.float32),
        pltpu.VMEM((RBLK * F,), jnp.float32),
        pltpu.VMEM((RBLK * F,), jnp.float32),
        pltpu.VMEM((LD_PER_W,), jnp.float32),
        pltpu.SemaphoreType.DMA,
        pltpu.SemaphoreType.DMA,
        pltpu.SemaphoreType.DMA,
        pltpu.SemaphoreType.DMA,
    ],
)
def _sc_permute(x_hbm, perm_hbm, out_hbm, ld_hbm, perm_v,
                inb0, inb1, outb0, outb1, ldb, is0, is1, os0, os1):
    wid = lax.axis_index("s") * NC + lax.axis_index("c")
    in_base = B_TC + wid * ROWS_PER_W
    out_base = wid * ROWS_PER_W
    inbufs, outbufs = [inb0, inb1], [outb0, outb1]
    isems, osems = [is0, is1], [os0, os1]

    pltpu.sync_copy(perm_hbm, perm_v)

    zero = jnp.zeros((L,), jnp.float32)

    def zero_body(t, carry):
        ldb[pl.ds(t * L, L)] = zero
        return carry

    lax.fori_loop(0, LD_PER_W // L, zero_body, 0)
    pltpu.sync_copy(ldb, ld_hbm.at[pl.ds(wid * LD_PER_W, LD_PER_W)])

    def in_slice(gb):
        return x_hbm.at[pl.ds((in_base + gb * RBLK) * F, RBLK * F)]

    def out_slice(gb):
        return out_hbm.at[pl.ds((out_base + gb * RBLK) * F, RBLK * F)]

    for b in range(NBUF):
        pltpu.async_copy(in_slice(b), inbufs[b], isems[b])

    def block_body(g, carry):
        for b in range(NBUF):
            gb = g * NBUF + b
            pltpu.make_async_copy(in_slice(gb), inbufs[b], isems[b]).wait()

            @pl.when(gb >= NBUF)
            def _wait_out():
                pltpu.make_async_copy(
                    outbufs[b], out_slice(gb), osems[b]).wait()

            inb, outb = inbufs[b], outbufs[b]

            @plsc.parallel_loop(0, NCHUNK, unroll=4)
            def chunk_body(jc):
                j0 = jc * L
                pc = perm_v[pl.ds(j0, L)]
                for r in range(RBLK):
                    idx = pc + jnp.full((L,), r * F, jnp.int32)
                    outb[pl.ds(r * F + j0, L)] = plsc.load_gather(inb, [idx])

            pltpu.async_copy(outb, out_slice(gb), osems[b])

            @pl.when(gb + NBUF < NBLK)
            def _next_in():
                pltpu.async_copy(in_slice(gb + NBUF), inbufs[b], isems[b])
        return carry

    lax.fori_loop(0, NBLK // NBUF, block_body, 0)

    for b in range(NBUF):
        pltpu.make_async_copy(
            outbufs[b], out_slice(NBLK - NBUF + b), osems[b]).wait()


def _build_p_body(perm_ref, p_ref):
    rows = lax.broadcasted_iota(jnp.int32, (F, F), 0)
    cols = perm_ref[...][None, :]
    p_ref[...] = jnp.where(rows == cols, 1.0, 0.0).astype(jnp.bfloat16)


_build_p = pl.pallas_call(
    _build_p_body,
    out_shape=jax.ShapeDtypeStruct((F, F), jnp.bfloat16),
)

MBLK = 256


def _tc_matmul_body(x_ref, p_ref, out_ref):
    out_ref[...] = jnp.dot(
        x_ref[...].astype(jnp.bfloat16), p_ref[...],
        preferred_element_type=jnp.float32)


_tc_matmul = pl.pallas_call(
    _tc_matmul_body,
    grid=(B_TC // MBLK,),
    in_specs=[
        pl.BlockSpec((MBLK, F), lambda i: (i, 0)),
        pl.BlockSpec((F, F), lambda i: (0, 0)),
    ],
    out_specs=pl.BlockSpec((MBLK, F), lambda i: (i, 0)),
    out_shape=jax.ShapeDtypeStruct((B_TC, F), jnp.float32),
)


def kernel(x, perm, inv_perm):
    del inv_perm
    perm_i32 = perm.astype(jnp.int32)
    sc_rows_flat, log_det = _sc_permute(x.reshape(-1), perm_i32)
    p = _build_p(perm_i32)
    tc_rows = _tc_matmul(x[:B_TC], p)
    out = jnp.concatenate([tc_rows, sc_rows_flat.reshape(B_SC, F)], axis=0)
    return (out, log_det)


# speedup vs baseline: 4.9805x; 4.7410x over previous
import functools

import jax
import jax.numpy as jnp
from jax import lax
from jax.experimental import pallas as pl
from jax.experimental.pallas import tpu as pltpu
from jax.experimental.pallas import tpu_sc as plsc

B = 16384
F = 4096
L = 16
FH = F // 2

_info = plsc.get_sparse_core_info()
NC = _info.num_cores
NS = _info.num_subcores
NW = NC * NS

ROWS_PER_W = B // NW
RBLK = 8
NBUF = 2
NBLK = ROWS_PER_W // RBLK
NCHUNK_H = FH // L

_mesh = plsc.VectorSubcoreMesh(core_axis_name="c", subcore_axis_name="s")


@functools.partial(
    pl.kernel,
    mesh=_mesh,
    compiler_params=pltpu.CompilerParams(needs_layout_passes=False),
    out_type=(
        jax.ShapeDtypeStruct((B, F), jnp.float32),
        jax.ShapeDtypeStruct((B,), jnp.float32),
    ),
    scratch_types=[
        pltpu.VMEM((F,), jnp.int32),
        pltpu.VMEM((RBLK, F), jnp.float32),
        pltpu.VMEM((RBLK, F), jnp.float32),
        pltpu.VMEM((RBLK, FH), jnp.float32),
        pltpu.VMEM((RBLK, FH), jnp.float32),
        pltpu.VMEM((ROWS_PER_W,), jnp.float32),
        pltpu.SemaphoreType.DMA,
        pltpu.SemaphoreType.DMA,
        pltpu.SemaphoreType.DMA,
        pltpu.SemaphoreType.DMA,
    ],
)
def _sc_permute(x_hbm, perm_hbm, out_hbm, ld_hbm, perm_v,
                inb0, inb1, outb0, outb1, ldb, is0, is1, os0, os1):
    wid = lax.axis_index("s") * NC + lax.axis_index("c")
    base = wid * ROWS_PER_W
    inbufs, outbufs = [inb0, inb1], [outb0, outb1]
    isems, osems = [is0, is1], [os0, os1]

    pltpu.sync_copy(perm_hbm, perm_v)

    zero = jnp.zeros((L,), jnp.float32)

    def zero_body(t, carry):
        ldb[pl.ds(t * L, L)] = zero
        return carry

    lax.fori_loop(0, ROWS_PER_W // L, zero_body, 0)
    pltpu.sync_copy(ldb, ld_hbm.at[pl.ds(base, ROWS_PER_W)])

    def in_slice(g):
        return x_hbm.at[pl.ds(base + g * RBLK, RBLK)]

    def out_slice(g, h):
        return out_hbm.at[pl.ds(base + g * RBLK, RBLK), pl.ds(h * FH, FH)]

    for b in range(NBUF):
        pltpu.async_copy(in_slice(b), inbufs[b], isems[b])

    def block_body(m, carry):
        for b in range(NBUF):
            g = m * NBUF + b
            pltpu.make_async_copy(in_slice(g), inbufs[b], isems[b]).wait()
            inb = inbufs[b]

            for h in range(2):
                @pl.when(g >= 1)
                def _wait_out():
                    pltpu.make_async_copy(
                        outbufs[h], out_slice(g - 1, h), osems[h]).wait()

                outb = outbufs[h]

                @plsc.parallel_loop(0, NCHUNK_H, unroll=4)
                def chunk_body(jc):
                    pc = perm_v[pl.ds(h * FH + jc * L, L)]
                    for r in range(RBLK):
                        ridx = jnp.full((L,), r, jnp.int32)
                        outb[r, pl.ds(jc * L, L)] = plsc.load_gather(
                            inb, [ridx, pc])

                pltpu.async_copy(outb, out_slice(g, h), osems[h])

            @pl.when(g + NBUF < NBLK)
            def _next_in():
                pltpu.async_copy(in_slice(g + NBUF), inbufs[b], isems[b])
        return carry

    lax.fori_loop(0, NBLK // NBUF, block_body, 0)

    for h in range(2):
        pltpu.make_async_copy(
            outbufs[h], out_slice(NBLK - 1, h), osems[h]).wait()


def kernel(x, perm, inv_perm):
    del inv_perm
    out, log_det = _sc_permute(x, perm.astype(jnp.int32))
    return (out, log_det)
